# baseline (speedup 1.0000x reference)
import functools

import jax
import jax.numpy as jnp
from jax.experimental import pallas as pl
from jax.experimental.pallas import tpu as pltpu

BEAMS = 128
HID = 1024
VOCAB = 100000
K = 8
BV = 2048
NBLK = VOCAB // BV
MAIN = NBLK * BV
TAIL = VOCAB - MAIN
LANES = 128
NTILE = BV // LANES
TAILTILES = (TAIL + LANES - 1) // LANES
TAILPAD = TAILTILES * LANES

NEG = -1e30
BIGI = 2**30


def _bitonic_merge_desc(xs):
    n = len(xs)
    if n == 1:
        return xs
    half = n // 2
    hi = [jnp.maximum(xs[i], xs[i + half]) for i in range(half)]
    lo = [jnp.minimum(xs[i], xs[i + half]) for i in range(half)]
    return _bitonic_merge_desc(hi) + _bitonic_merge_desc(lo)


def _merge_desc(a, b):
    return _bitonic_merge_desc(a + b[::-1])


def _merge_top8(a, b):
    m = [jnp.maximum(a[i], b[7 - i]) for i in range(8)]
    return _bitonic_merge_desc(m)


def _block_top8(tiles):
    lists = [[t] for t in tiles]
    while len(lists) > 2:
        lists = [_merge_desc(lists[t], lists[t + 1])
                 for t in range(0, len(lists), 2)]
    return _merge_top8(lists[0], lists[1])


def _bitonic_merge_desc_kv(vs, ids):
    n = len(vs)
    if n == 1:
        return vs, ids
    half = n // 2
    hv, hi, lv, li = [], [], [], []
    for i in range(half):
        av, ai, bv, bi = vs[i], ids[i], vs[i + half], ids[i + half]
        c = (bv > av) | ((bv == av) & (bi < ai))
        hv.append(jnp.where(c, bv, av))
        hi.append(jnp.where(c, bi, ai))
        lv.append(jnp.where(c, av, bv))
        li.append(jnp.where(c, ai, bi))
    rhv, rhi = _bitonic_merge_desc_kv(hv, hi)
    rlv, rli = _bitonic_merge_desc_kv(lv, li)
    return rhv + rlv, rhi + rli


def _merge_top8_kv(av, ai, bv, bi):
    mv, mi = [], []
    for i in range(8):
        x, xi_, y, yi = av[i], ai[i], bv[7 - i], bi[7 - i]
        c = (y > x) | ((y == x) & (yi < xi_))
        mv.append(jnp.where(c, y, x))
        mi.append(jnp.where(c, yi, xi_))
    return _bitonic_merge_desc_kv(mv, mi)


def _extract_topk(x, ids, k):
    vs, is_ = [], []
    for _ in range(k):
        m = jnp.max(x, axis=1, keepdims=True)
        sel = jnp.min(jnp.where(x == m, ids, BIGI), axis=1, keepdims=True)
        vs.append(m)
        is_.append(sel)
        x = jnp.where(ids == sel, NEG, x)
    return jnp.concatenate(vs, axis=1), jnp.concatenate(is_, axis=1)


def _pack_tiles(x, width):
    tiles = []
    for i in range(width // LANES):
        xi = jax.lax.bitcast_convert_type(
            x[:, i * LANES:(i + 1) * LANES], jnp.int32)
        tiles.append(jax.lax.bitcast_convert_type((xi & -16) | i,
                                                  jnp.float32))
    return tiles


def _unpack_top8(blk, base):
    lane = jax.lax.broadcasted_iota(jnp.int32, (BEAMS, LANES), 1)
    bv_, bi_ = [], []
    for s in range(K):
        y = jax.lax.bitcast_convert_type(blk[s], jnp.int32)
        bi_.append((y & 15) * LANES + lane + base)
        bv_.append(jax.lax.bitcast_convert_type(y & -16, jnp.float32))
    return bv_, bi_


def _update_stats(x, m_scr, s_scr):
    m_old = m_scr[...]
    bm = jnp.max(x, axis=1, keepdims=True)
    m_new = jnp.maximum(m_old, bm)
    s_scr[...] = (s_scr[...] * jnp.exp(m_old - m_new)
                  + jnp.sum(jnp.exp(x - m_new), axis=1, keepdims=True))
    m_scr[...] = m_new


def _merge_into_planes(bv_, bi_, pv_scr, pi_scr):
    pv = [pv_scr[:, p * LANES:(p + 1) * LANES] for p in range(K)]
    pi = [pi_scr[:, p * LANES:(p + 1) * LANES] for p in range(K)]
    nv, ni = _merge_top8_kv(pv, pi, bv_, bi_)
    for p in range(K):
        pv_scr[:, p * LANES:(p + 1) * LANES] = nv[p]
        pi_scr[:, p * LANES:(p + 1) * LANES] = ni[p]


def _step(hid_ref, w_ref, b_ref, prev_ref, tail_ref,
          vals_out, ids_out,
          m_scr, s_scr, pv_scr, pi_scr):
    j = pl.program_id(0)

    @pl.when(j == 0)
    def _init():
        m_scr[...] = jnp.full((BEAMS, 1), NEG, jnp.float32)
        s_scr[...] = jnp.zeros((BEAMS, 1), jnp.float32)
        pv_scr[...] = jnp.full((BEAMS, K * LANES), NEG, jnp.float32)
        pi_scr[...] = jnp.full((BEAMS, K * LANES), BIGI, jnp.int32)

    x = jax.lax.dot_general(
        hid_ref[...], w_ref[...], (((1,), (0,)), ((), ())),
        preferred_element_type=jnp.float32,
        precision=jax.lax.Precision.HIGHEST,
    ) + b_ref[...]

    _update_stats(x, m_scr, s_scr)
    bv_, bi_ = _unpack_top8(_block_top8(_pack_tiles(x, BV)), j * BV)
    _merge_into_planes(bv_, bi_, pv_scr, pi_scr)

    @pl.when(j == NBLK - 1)
    def _finalize():
        t = tail_ref[...]
        _update_stats(t, m_scr, s_scr)
        ttiles = _pack_tiles(t, TAILPAD)
        negtile = jnp.full((BEAMS, LANES), NEG, jnp.float32)
        ttiles += [negtile] * (NTILE - TAILTILES)
        tbv, tbi = _unpack_top8(_block_top8(ttiles), MAIN)
        _merge_into_planes(tbv, tbi, pv_scr, pi_scr)

        tv, ti = _extract_topk(pv_scr[...], pi_scr[...], K)
        lse = m_scr[...] + jnp.log(s_scr[...])
        sc = prev_ref[...] + tv - lse
        row = jax.lax.broadcasted_iota(jnp.int32, (BEAMS, K), 0)
        flat = row * VOCAB + ti
        ocol = jax.lax.broadcasted_iota(jnp.int32, (1, K), 1)
        ov = jnp.zeros((1, K), jnp.float32)
        oi = jnp.zeros((1, K), jnp.int32)
        for r in range(K):
            m = jnp.max(sc, axis=(0, 1), keepdims=True)
            chosen = jnp.min(jnp.where(sc == m, flat, BIGI),
                             axis=(0, 1), keepdims=True)
            ov = jnp.where(ocol == r, m, ov)
            oi = jnp.where(ocol == r, chosen, oi)
            sc = jnp.where(flat == chosen, NEG, sc)
        vals_out[...] = ov
        ids_out[...] = oi


@functools.partial(jax.jit, static_argnames=())
def kernel(hidden, W, b, prev_log_probs):
    b2 = b.reshape(1, VOCAB)
    prev2 = prev_log_probs.reshape(BEAMS, 1)
    tail = jax.lax.dot_general(
        hidden, jax.lax.slice(W, (0, MAIN), (HID, VOCAB)),
        (((1,), (0,)), ((), ())),
        preferred_element_type=jnp.float32,
        precision=jax.lax.Precision.HIGHEST,
    ) + b[MAIN:]
    tail = jnp.concatenate(
        [tail, jnp.full((BEAMS, TAILPAD - TAIL), NEG, jnp.float32)], axis=1)
    vals, flat = pl.pallas_call(
        _step,
        grid=(NBLK,),
        in_specs=[
            pl.BlockSpec((BEAMS, HID), lambda j: (0, 0)),
            pl.BlockSpec((HID, BV), lambda j: (0, j)),
            pl.BlockSpec((1, BV), lambda j: (0, j)),
            pl.BlockSpec((BEAMS, 1), lambda j: (0, 0)),
            pl.BlockSpec((BEAMS, TAILPAD), lambda j: (0, 0)),
        ],
        out_specs=[
            pl.BlockSpec((1, K), lambda j: (0, 0)),
            pl.BlockSpec((1, K), lambda j: (0, 0)),
        ],
        out_shape=[
            jax.ShapeDtypeStruct((1, K), jnp.float32),
            jax.ShapeDtypeStruct((1, K), jnp.int32),
        ],
        scratch_shapes=[
            pltpu.VMEM((BEAMS, 1), jnp.float32),
            pltpu.VMEM((BEAMS, 1), jnp.float32),
            pltpu.VMEM((BEAMS, K * LANES), jnp.float32),
            pltpu.VMEM((BEAMS, K * LANES), jnp.int32),
        ],
        compiler_params=pltpu.CompilerParams(
            dimension_semantics=("arbitrary",),
        ),
    )(hidden, W, b2, prev2, tail)
    vals = vals.reshape(K)
    flat = flat.reshape(K)
    beam_ids = flat // VOCAB
    token_ids = flat % VOCAB
    return vals, beam_ids, token_ids


# speedup vs baseline: 1.3020x; 1.3020x over previous
import functools

import jax
import jax.numpy as jnp
from jax.experimental import pallas as pl
from jax.experimental.pallas import tpu as pltpu

BEAMS = 128
HID = 1024
VOCAB = 100000
K = 8
BV = 2000
NBLK = VOCAB // BV
LANES = 128
NTILE = (BV + LANES - 1) // LANES

NEG = -1e30
BIGI = 2**30


def _bitonic_merge_desc(xs):
    n = len(xs)
    if n == 1:
        return xs
    half = n // 2
    hi = [jnp.maximum(xs[i], xs[i + half]) for i in range(half)]
    lo = [jnp.minimum(xs[i], xs[i + half]) for i in range(half)]
    return _bitonic_merge_desc(hi) + _bitonic_merge_desc(lo)


def _merge_desc(a, b):
    return _bitonic_merge_desc(a + b[::-1])


def _merge_top8(a, b):
    m = [jnp.maximum(a[i], b[7 - i]) for i in range(8)]
    return _bitonic_merge_desc(m)


def _block_top8(tiles):
    lists = [[t] for t in tiles]
    while len(lists) > 2:
        lists = [_merge_desc(lists[t], lists[t + 1])
                 for t in range(0, len(lists), 2)]
    return _merge_top8(lists[0], lists[1])


def _bitonic_merge_desc_kv(vs, ids):
    n = len(vs)
    if n == 1:
        return vs, ids
    half = n // 2
    hv, hi, lv, li = [], [], [], []
    for i in range(half):
        av, ai, bv, bi = vs[i], ids[i], vs[i + half], ids[i + half]
        c = (bv > av) | ((bv == av) & (bi < ai))
        hv.append(jnp.where(c, bv, av))
        hi.append(jnp.where(c, bi, ai))
        lv.append(jnp.where(c, av, bv))
        li.append(jnp.where(c, ai, bi))
    rhv, rhi = _bitonic_merge_desc_kv(hv, hi)
    rlv, rli = _bitonic_merge_desc_kv(lv, li)
    return rhv + rlv, rhi + rli


def _merge_top8_kv(av, ai, bv, bi):
    mv, mi = [], []
    for i in range(8):
        x, xi_, y, yi = av[i], ai[i], bv[7 - i], bi[7 - i]
        c = (y > x) | ((y == x) & (yi < xi_))
        mv.append(jnp.where(c, y, x))
        mi.append(jnp.where(c, yi, xi_))
    return _bitonic_merge_desc_kv(mv, mi)


def _extract_topk(x, ids, k):
    vs, is_ = [], []
    for _ in range(k):
        m = jnp.max(x, axis=1, keepdims=True)
        sel = jnp.min(jnp.where(x == m, ids, BIGI), axis=1, keepdims=True)
        vs.append(m)
        is_.append(sel)
        x = jnp.where(ids == sel, NEG, x)
    return jnp.concatenate(vs, axis=1), jnp.concatenate(is_, axis=1)


def _step(hid_ref, wt_ref, b_ref, prev_ref,
          vals_out, ids_out,
          m_scr, s_scr, pv_scr, pi_scr):
    j = pl.program_id(0)

    @pl.when(j == 0)
    def _init():
        m_scr[...] = jnp.full((BEAMS, 1), NEG, jnp.float32)
        s_scr[...] = jnp.zeros((BEAMS, 1), jnp.float32)
        pv_scr[...] = jnp.full((BEAMS, K * LANES), NEG, jnp.float32)
        pi_scr[...] = jnp.full((BEAMS, K * LANES), BIGI, jnp.int32)

    x = jax.lax.dot_general(
        hid_ref[...], wt_ref[...], (((1,), (1,)), ((), ())),
        preferred_element_type=jnp.float32,
        precision=jax.lax.Precision.HIGHEST,
    ) + b_ref[...].reshape(1, BV)

    m_old = m_scr[...]
    bm = jnp.max(x, axis=1, keepdims=True)
    m_new = jnp.maximum(m_old, bm)
    s_scr[...] = (s_scr[...] * jnp.exp(m_old - m_new)
                  + jnp.sum(jnp.exp(x - m_new), axis=1, keepdims=True))
    m_scr[...] = m_new

    tiles = []
    for i in range(NTILE):
        xt = x[:, i * LANES:min((i + 1) * LANES, BV)]
        if xt.shape[1] < LANES:
            xt = jnp.concatenate(
                [xt, jnp.full((BEAMS, LANES - xt.shape[1]), NEG,
                              jnp.float32)], axis=1)
        xi = jax.lax.bitcast_convert_type(xt, jnp.int32)
        tiles.append(jax.lax.bitcast_convert_type((xi & -16) | i,
                                                  jnp.float32))
    blk = _block_top8(tiles)

    lane = jax.lax.broadcasted_iota(jnp.int32, (BEAMS, LANES), 1)
    bv_, bi_ = [], []
    for s in range(K):
        y = jax.lax.bitcast_convert_type(blk[s], jnp.int32)
        bi_.append((y & 15) * LANES + lane + j * BV)
        bv_.append(jax.lax.bitcast_convert_type(y & -16, jnp.float32))

    pv = [pv_scr[:, p * LANES:(p + 1) * LANES] for p in range(K)]
    pi = [pi_scr[:, p * LANES:(p + 1) * LANES] for p in range(K)]
    nv, ni = _merge_top8_kv(pv, pi, bv_, bi_)
    for p in range(K):
        pv_scr[:, p * LANES:(p + 1) * LANES] = nv[p]
        pi_scr[:, p * LANES:(p + 1) * LANES] = ni[p]

    @pl.when(j == NBLK - 1)
    def _finalize():
        tv, ti = _extract_topk(pv_scr[...], pi_scr[...], K)
        lse = m_scr[...] + jnp.log(s_scr[...])
        sc = prev_ref[...] + tv - lse
        row = jax.lax.broadcasted_iota(jnp.int32, (BEAMS, K), 0)
        flat = row * VOCAB + ti
        ocol = jax.lax.broadcasted_iota(jnp.int32, (1, K), 1)
        ov = jnp.zeros((1, K), jnp.float32)
        oi = jnp.zeros((1, K), jnp.int32)
        for r in range(K):
            m = jnp.max(sc, axis=(0, 1), keepdims=True)
            chosen = jnp.min(jnp.where(sc == m, flat, BIGI),
                             axis=(0, 1), keepdims=True)
            ov = jnp.where(ocol == r, m, ov)
            oi = jnp.where(ocol == r, chosen, oi)
            sc = jnp.where(flat == chosen, NEG, sc)
        vals_out[...] = ov
        ids_out[...] = oi


@functools.partial(jax.jit, static_argnames=())
def kernel(hidden, W, b, prev_log_probs):
    wt = W.T
    b2 = b.reshape(NBLK, 1, BV)
    prev2 = prev_log_probs.reshape(BEAMS, 1)
    vals, flat = pl.pallas_call(
        _step,
        grid=(NBLK,),
        in_specs=[
            pl.BlockSpec((BEAMS, HID), lambda j: (0, 0)),
            pl.BlockSpec((BV, HID), lambda j: (j, 0)),
            pl.BlockSpec((1, 1, BV), lambda j: (j, 0, 0)),
            pl.BlockSpec((BEAMS, 1), lambda j: (0, 0)),
        ],
        out_specs=[
            pl.BlockSpec((1, K), lambda j: (0, 0)),
            pl.BlockSpec((1, K), lambda j: (0, 0)),
        ],
        out_shape=[
            jax.ShapeDtypeStruct((1, K), jnp.float32),
            jax.ShapeDtypeStruct((1, K), jnp.int32),
        ],
        scratch_shapes=[
            pltpu.VMEM((BEAMS, 1), jnp.float32),
            pltpu.VMEM((BEAMS, 1), jnp.float32),
            pltpu.VMEM((BEAMS, K * LANES), jnp.float32),
            pltpu.VMEM((BEAMS, K * LANES), jnp.int32),
        ],
        compiler_params=pltpu.CompilerParams(
            dimension_semantics=("arbitrary",),
        ),
    )(hidden, wt, b2, prev2)
    vals = vals.reshape(K)
    flat = flat.reshape(K)
    beam_ids = flat // VOCAB
    token_ids = flat % VOCAB
    return vals, beam_ids, token_ids


# speedup vs baseline: 1.4071x; 1.0807x over previous
import functools

import jax
import jax.numpy as jnp
from jax.experimental import pallas as pl
from jax.experimental.pallas import tpu as pltpu

BEAMS = 128
HID = 1024
VOCAB = 100000
K = 8
BV = 2000
NBLK = VOCAB // BV
LANES = 128
NSLOT = BV // 8
NGRP = 16

NEG = -1e30
BIGI = 2**30


def _bitonic_merge_desc(xs):
    n = len(xs)
    if n == 1:
        return xs
    half = n // 2
    hi = [jnp.maximum(xs[i], xs[i + half]) for i in range(half)]
    lo = [jnp.minimum(xs[i], xs[i + half]) for i in range(half)]
    return _bitonic_merge_desc(hi) + _bitonic_merge_desc(lo)


def _merge_desc(a, b):
    return _bitonic_merge_desc(a + b[::-1])


def _merge_top8(a, b):
    m = [jnp.maximum(a[i], b[7 - i]) for i in range(8)]
    return _bitonic_merge_desc(m)


def _block_top8(tiles):
    lists = [[t] for t in tiles]
    while len(lists) > 2:
        lists = [_merge_desc(lists[t], lists[t + 1])
                 for t in range(0, len(lists), 2)]
    return _merge_top8(lists[0], lists[1])


def _bitonic_merge_desc_kv(vs, ids):
    n = len(vs)
    if n == 1:
        return vs, ids
    half = n // 2
    hv, hi, lv, li = [], [], [], []
    for i in range(half):
        av, ai, bv, bi = vs[i], ids[i], vs[i + half], ids[i + half]
        c = (bv > av) | ((bv == av) & (bi < ai))
        hv.append(jnp.where(c, bv, av))
        hi.append(jnp.where(c, bi, ai))
        lv.append(jnp.where(c, av, bv))
        li.append(jnp.where(c, ai, bi))
    rhv, rhi = _bitonic_merge_desc_kv(hv, hi)
    rlv, rli = _bitonic_merge_desc_kv(lv, li)
    return rhv + rlv, rhi + rli


def _merge_top8_kv(av, ai, bv, bi):
    mv, mi = [], []
    for i in range(8):
        x, xi_, y, yi = av[i], ai[i], bv[7 - i], bi[7 - i]
        c = (y > x) | ((y == x) & (yi < xi_))
        mv.append(jnp.where(c, y, x))
        mi.append(jnp.where(c, yi, xi_))
    return _bitonic_merge_desc_kv(mv, mi)


def _step(hidt_ref, wt_ref, b_ref, prev_ref,
          vals_out, ids_out,
          m_scr, s_scr, pv_scr, pi_scr):
    j = pl.program_id(0)

    @pl.when(j == 0)
    def _init():
        m_scr[...] = jnp.full((1, LANES), NEG, jnp.float32)
        s_scr[...] = jnp.zeros((1, LANES), jnp.float32)
        pv_scr[...] = jnp.full((8 * K, LANES), NEG, jnp.float32)
        pi_scr[...] = jnp.full((8 * K, LANES), BIGI, jnp.int32)

    ones = jnp.full((1, LANES), 1.0, jnp.float32)
    bias = jax.lax.dot_general(
        b_ref[...].reshape(1, BV), ones, (((0,), (0,)), ((), ())),
        preferred_element_type=jnp.float32,
        precision=jax.lax.Precision.HIGHEST,
    )
    xt = jax.lax.dot_general(
        wt_ref[...], hidt_ref[...], (((1,), (0,)), ((), ())),
        preferred_element_type=jnp.float32,
        precision=jax.lax.Precision.HIGHEST,
    ) + bias

    m_old = m_scr[...]
    bm = jnp.max(xt, axis=0, keepdims=True)
    m_new = jnp.maximum(m_old, bm)
    s_scr[...] = (s_scr[...] * jnp.exp(m_old - m_new)
                  + jnp.sum(jnp.exp(xt - m_new), axis=0, keepdims=True))
    m_scr[...] = m_new

    subl = jax.lax.broadcasted_iota(jnp.int32, (8, LANES), 0)
    negslot = jnp.full((8, LANES), NEG, jnp.float32)
    gv, gi = [], []
    for g in range(NGRP):
        tiles = []
        for t in range(16):
            s = g * 16 + t
            if s < NSLOT:
                xi = jax.lax.bitcast_convert_type(
                    xt[s * 8:(s + 1) * 8, :], jnp.int32)
                tiles.append(jax.lax.bitcast_convert_type(
                    (xi & -16) | t, jnp.float32))
            else:
                tiles.append(negslot)
        blk = _block_top8(tiles)
        bv_, bi_ = [], []
        for r in range(K):
            y = jax.lax.bitcast_convert_type(blk[r], jnp.int32)
            slot = (y & 15) + g * 16
            bi_.append(slot * 8 + subl + j * BV)
            bv_.append(jax.lax.bitcast_convert_type(y & -16, jnp.float32))
        gv.append(bv_)
        gi.append(bi_)

    while len(gv) > 1:
        nv, ni = [], []
        for t in range(0, len(gv), 2):
            mv, mi = _merge_top8_kv(gv[t], gi[t], gv[t + 1], gi[t + 1])
            nv.append(mv)
            ni.append(mi)
        gv, gi = nv, ni

    pv = [pv_scr[p * 8:(p + 1) * 8, :] for p in range(K)]
    pi = [pi_scr[p * 8:(p + 1) * 8, :] for p in range(K)]
    nv, ni = _merge_top8_kv(pv, pi, gv[0], gi[0])
    for p in range(K):
        pv_scr[p * 8:(p + 1) * 8, :] = nv[p]
        pi_scr[p * 8:(p + 1) * 8, :] = ni[p]

    @pl.when(j == NBLK - 1)
    def _finalize():
        x = pv_scr[...]
        ids = pi_scr[...]
        tvs, tis = [], []
        for _ in range(K):
            m = jnp.max(x, axis=0, keepdims=True)
            sel = jnp.min(jnp.where(x == m, ids, BIGI), axis=0,
                          keepdims=True)
            tvs.append(m)
            tis.append(sel)
            x = jnp.where(ids == sel, NEG, x)
        tv = jnp.concatenate(tvs, axis=0)
        ti = jnp.concatenate(tis, axis=0)
        lse = m_scr[...] + jnp.log(s_scr[...])
        sc = prev_ref[...] + tv - lse
        beam = jax.lax.broadcasted_iota(jnp.int32, (K, LANES), 1)
        flat = beam * VOCAB + ti
        ocol = jax.lax.broadcasted_iota(jnp.int32, (1, K), 1)
        ov = jnp.zeros((1, K), jnp.float32)
        oi = jnp.zeros((1, K), jnp.int32)
        for r in range(K):
            m = jnp.max(sc, axis=(0, 1), keepdims=True)
            chosen = jnp.min(jnp.where(sc == m, flat, BIGI),
                             axis=(0, 1), keepdims=True)
            ov = jnp.where(ocol == r, m, ov)
            oi = jnp.where(ocol == r, chosen, oi)
            sc = jnp.where(flat == chosen, NEG, sc)
        vals_out[...] = ov
        ids_out[...] = oi


@functools.partial(jax.jit, static_argnames=())
def kernel(hidden, W, b, prev_log_probs):
    wt = W.T
    hidt = hidden.T
    b2 = b.reshape(NBLK, 1, BV)
    prev2 = prev_log_probs.reshape(1, BEAMS)
    vals, flat = pl.pallas_call(
        _step,
        grid=(NBLK,),
        in_specs=[
            pl.BlockSpec((HID, BEAMS), lambda j: (0, 0)),
            pl.BlockSpec((BV, HID), lambda j: (j, 0)),
            pl.BlockSpec((1, 1, BV), lambda j: (j, 0, 0)),
            pl.BlockSpec((1, BEAMS), lambda j: (0, 0)),
        ],
        out_specs=[
            pl.BlockSpec((1, K), lambda j: (0, 0)),
            pl.BlockSpec((1, K), lambda j: (0, 0)),
        ],
        out_shape=[
            jax.ShapeDtypeStruct((1, K), jnp.float32),
            jax.ShapeDtypeStruct((1, K), jnp.int32),
        ],
        scratch_shapes=[
            pltpu.VMEM((1, LANES), jnp.float32),
            pltpu.VMEM((1, LANES), jnp.float32),
            pltpu.VMEM((8 * K, LANES), jnp.float32),
            pltpu.VMEM((8 * K, LANES), jnp.int32),
        ],
        compiler_params=pltpu.CompilerParams(
            dimension_semantics=("arbitrary",),
        ),
    )(hidt, wt, b2, prev2)
    vals = vals.reshape(K)
    flat = flat.reshape(K)
    beam_ids = flat // VOCAB
    token_ids = flat % VOCAB
    return vals, beam_ids, token_ids
